# baseline (speedup 1.0000x reference)
import functools

import jax
import jax.numpy as jnp
from jax.experimental import pallas as pl
from jax.experimental.pallas import tpu as pltpu


def _round_up(a, b):
    return (a + b - 1) // b * b


def _fused_kernel(scale_ref, shift_ref, x_ref,
                  we_ref, be_ref, ln1w_ref, ln1b_ref,
                  w1_ref, b1_ref, w2_ref, b2_ref, ln2w_ref, ln2b_ref,
                  m_ref, k_ref, o_ref, *, cliprange, eps=1e-5):
    bf = jnp.bfloat16
    f32 = jnp.float32
    x = x_ref[...]

    e = jnp.where(x == 0.0, 1.0, 0.0).astype(bf)
    keep = 1.0 - jnp.dot(e, k_ref[...], preferred_element_type=f32)

    xn = jnp.clip(x * scale_ref[...] - shift_ref[...], -cliprange, cliprange)

    m = m_ref[...]

    def ln(y, w, b):
        mu = jnp.dot(y.astype(bf), m, preferred_element_type=f32)
        yc = y - mu
        v = jnp.dot((yc * yc).astype(bf), m, preferred_element_type=f32)
        return yc * jax.lax.rsqrt(v + eps) * w + b

    h0 = jnp.dot(xn.astype(bf), we_ref[...], preferred_element_type=f32)
    h = ln(jnp.maximum(h0 + be_ref[...], 0.0), ln1w_ref[...], ln1b_ref[...])
    f0 = jnp.dot(h.astype(bf), w1_ref[...], preferred_element_type=f32)
    fr = jnp.maximum(f0 + b1_ref[...], 0.0)
    r0 = jnp.dot(fr.astype(bf), w2_ref[...], preferred_element_type=f32)
    rr = jnp.maximum(r0 + b2_ref[...], 0.0)
    h = ln(h + rr, ln2w_ref[...], ln2b_ref[...])
    o_ref[...] = (h * keep).astype(o_ref.dtype)


def kernel(x, mean, squares_sum, count, w_emb, b_emb, ln1_w, ln1_b,
           w_ff1, b_ff1, w_ff2, b_ff2, ln2_w, ln2_b, *, block_rows=4096):
    B, items, d_in = x.shape
    d_model = w_emb.shape[1]
    R = B * items
    x2 = x.reshape(R, d_in)

    tr = _round_up(min(block_rows, _round_up(R, 8)), 8)
    R_pad = _round_up(R, tr)
    if R_pad != R:
        x2 = jnp.pad(x2, ((0, R_pad - R), (0, 0)))

    bf = jnp.bfloat16
    count_f = count.astype(jnp.float32)
    denom = jnp.maximum(count_f - 1.0, 1.0)
    var0 = squares_sum.astype(jnp.float32) / denom
    inv_sd = jnp.where(var0 != 0.0, jax.lax.rsqrt(var0), 1.0)
    use_norm = count_f > 1.0
    scale = jnp.where(use_norm, inv_sd, 1.0)
    shift = jnp.where(use_norm, mean.astype(jnp.float32), 0.0) * scale
    scale_r = scale.reshape(1, d_in)
    shift_r = shift.reshape(1, d_in)

    m_mat = jnp.full((d_model, d_model), 1.0 / d_model, dtype=bf)
    k_sel = jnp.zeros((d_in, d_model), jnp.float32).at[0, :].set(1.0).astype(bf)

    weights = [w_emb.astype(bf), b_emb, ln1_w, ln1_b,
               w_ff1.astype(bf), b_ff1, w_ff2.astype(bf), b_ff2,
               ln2_w, ln2_b, m_mat, k_sel]
    weight_specs = [pl.BlockSpec(tuple(w.shape), lambda i: (0, 0))
                    for w in weights]

    out = pl.pallas_call(
        functools.partial(_fused_kernel, cliprange=5.0),
        out_shape=jax.ShapeDtypeStruct((R_pad, d_model), jnp.float32),
        grid=(R_pad // tr,),
        in_specs=[
            pl.BlockSpec((1, d_in), lambda i: (0, 0)),
            pl.BlockSpec((1, d_in), lambda i: (0, 0)),
            pl.BlockSpec((tr, d_in), lambda i: (i, 0)),
        ] + weight_specs,
        out_specs=pl.BlockSpec((tr, d_model), lambda i: (i, 0)),
        compiler_params=pltpu.CompilerParams(
            dimension_semantics=("parallel",),
            vmem_limit_bytes=64 * 1024 * 1024,
        ),
    )(scale_r, shift_r, x2, *weights)

    y = (out if R_pad == R else out[:R]).reshape(B, items, d_model)
    mask = x[:, :, 0] == 0
    return y, None, mask


# speedup vs baseline: 1.0170x; 1.0170x over previous
import functools

import jax
import jax.numpy as jnp
from jax.experimental import pallas as pl
from jax.experimental.pallas import tpu as pltpu


def _round_up(a, b):
    return (a + b - 1) // b * b


def _fused_kernel(scale_ref, shift_ref, x_ref,
                  we_ref, be_ref, ln1w_ref, ln1b_ref,
                  w1_ref, b1_ref, w2_ref, b2_ref, ln2w_ref, ln2b_ref,
                  m_ref, k_ref, o_ref, *, cliprange, eps=1e-5):
    bf = jnp.bfloat16
    f32 = jnp.float32
    x = x_ref[...]

    e = jnp.where(x == 0.0, 1.0, 0.0).astype(bf)
    keep = 1.0 - jnp.dot(e, k_ref[...], preferred_element_type=f32)

    xn = jnp.clip(x * scale_ref[...] - shift_ref[...], -cliprange, cliprange)

    m = m_ref[...]

    def ln(y, w, b):
        mu = jnp.dot(y.astype(bf), m, preferred_element_type=f32)
        yc = y - mu
        v = jnp.dot((yc * yc).astype(bf), m, preferred_element_type=f32)
        return yc * jax.lax.rsqrt(v + eps) * w + b

    h0 = jnp.dot(xn.astype(bf), we_ref[...], preferred_element_type=f32)
    h = ln(jnp.maximum(h0 + be_ref[...], 0.0), ln1w_ref[...], ln1b_ref[...])
    f0 = jnp.dot(h.astype(bf), w1_ref[...], preferred_element_type=f32)
    fr = jnp.maximum(f0 + b1_ref[...], 0.0)
    r0 = jnp.dot(fr.astype(bf), w2_ref[...], preferred_element_type=f32)
    rr = jnp.maximum(r0 + b2_ref[...], 0.0)
    h = ln(h + rr, ln2w_ref[...], ln2b_ref[...])
    o_ref[...] = (h * keep).astype(o_ref.dtype)


def kernel(x, mean, squares_sum, count, w_emb, b_emb, ln1_w, ln1_b,
           w_ff1, b_ff1, w_ff2, b_ff2, ln2_w, ln2_b, *, block_rows=8192):
    B, items, d_in = x.shape
    d_model = w_emb.shape[1]
    R = B * items
    x2 = x.reshape(R, d_in)

    tr = _round_up(min(block_rows, _round_up(R, 8)), 8)
    R_pad = _round_up(R, tr)
    if R_pad != R:
        x2 = jnp.pad(x2, ((0, R_pad - R), (0, 0)))

    bf = jnp.bfloat16
    count_f = count.astype(jnp.float32)
    denom = jnp.maximum(count_f - 1.0, 1.0)
    var0 = squares_sum.astype(jnp.float32) / denom
    inv_sd = jnp.where(var0 != 0.0, jax.lax.rsqrt(var0), 1.0)
    use_norm = count_f > 1.0
    scale = jnp.where(use_norm, inv_sd, 1.0)
    shift = jnp.where(use_norm, mean.astype(jnp.float32), 0.0) * scale
    scale_r = scale.reshape(1, d_in)
    shift_r = shift.reshape(1, d_in)

    m_mat = jnp.full((d_model, d_model), 1.0 / d_model, dtype=bf)
    k_sel = jnp.zeros((d_in, d_model), jnp.float32).at[0, :].set(1.0).astype(bf)

    weights = [w_emb.astype(bf), b_emb, ln1_w, ln1_b,
               w_ff1.astype(bf), b_ff1, w_ff2.astype(bf), b_ff2,
               ln2_w, ln2_b, m_mat, k_sel]
    weight_specs = [pl.BlockSpec(tuple(w.shape), lambda i: (0, 0))
                    for w in weights]

    out = pl.pallas_call(
        functools.partial(_fused_kernel, cliprange=5.0),
        out_shape=jax.ShapeDtypeStruct((R_pad, d_model), jnp.float32),
        grid=(R_pad // tr,),
        in_specs=[
            pl.BlockSpec((1, d_in), lambda i: (0, 0)),
            pl.BlockSpec((1, d_in), lambda i: (0, 0)),
            pl.BlockSpec((tr, d_in), lambda i: (i, 0)),
        ] + weight_specs,
        out_specs=pl.BlockSpec((tr, d_model), lambda i: (i, 0)),
        compiler_params=pltpu.CompilerParams(
            dimension_semantics=("parallel",),
            vmem_limit_bytes=64 * 1024 * 1024,
        ),
    )(scale_r, shift_r, x2, *weights)

    y = (out if R_pad == R else out[:R]).reshape(B, items, d_model)
    mask = x[:, :, 0] == 0
    return y, None, mask


# speedup vs baseline: 1.0181x; 1.0011x over previous
import functools

import jax
import jax.numpy as jnp
from jax.experimental import pallas as pl
from jax.experimental.pallas import tpu as pltpu


def _round_up(a, b):
    return (a + b - 1) // b * b


def _fused_kernel(scale_ref, shift_ref, x_ref,
                  we_ref, be_ref, ln1w_ref, ln1b_ref,
                  w1_ref, b1_ref, w2_ref, b2_ref, ln2w_ref, ln2b_ref,
                  m_ref, k_ref, o_ref, *, cliprange, eps=1e-5):
    bf = jnp.bfloat16
    f32 = jnp.float32
    x = x_ref[...]

    e = jnp.where(x == 0.0, 1.0, 0.0).astype(bf)
    keep = 1.0 - jnp.dot(e, k_ref[...], preferred_element_type=f32)

    xn = jnp.clip(x * scale_ref[...] - shift_ref[...], -cliprange, cliprange)

    m = m_ref[...]

    def ln(y, w, b):
        mu = jnp.dot(y.astype(bf), m, preferred_element_type=f32)
        yc = y - mu
        v = jnp.dot((yc * yc).astype(bf), m, preferred_element_type=f32)
        return yc * jax.lax.rsqrt(v + eps) * w + b

    h0 = jnp.dot(xn.astype(bf), we_ref[...], preferred_element_type=f32)
    h = ln(jnp.maximum(h0 + be_ref[...], 0.0), ln1w_ref[...], ln1b_ref[...])
    f0 = jnp.dot(h.astype(bf), w1_ref[...], preferred_element_type=f32)
    fr = jnp.maximum(f0 + b1_ref[...], 0.0)
    r0 = jnp.dot(fr.astype(bf), w2_ref[...], preferred_element_type=f32)
    rr = jnp.maximum(r0 + b2_ref[...], 0.0)
    h = ln(h + rr, ln2w_ref[...], ln2b_ref[...])
    o_ref[...] = (h * keep).astype(o_ref.dtype)


def kernel(x, mean, squares_sum, count, w_emb, b_emb, ln1_w, ln1_b,
           w_ff1, b_ff1, w_ff2, b_ff2, ln2_w, ln2_b, *, block_rows=16384):
    B, items, d_in = x.shape
    d_model = w_emb.shape[1]
    R = B * items
    x2 = x.reshape(R, d_in)

    tr = _round_up(min(block_rows, _round_up(R, 8)), 8)
    R_pad = _round_up(R, tr)
    if R_pad != R:
        x2 = jnp.pad(x2, ((0, R_pad - R), (0, 0)))

    bf = jnp.bfloat16
    count_f = count.astype(jnp.float32)
    denom = jnp.maximum(count_f - 1.0, 1.0)
    var0 = squares_sum.astype(jnp.float32) / denom
    inv_sd = jnp.where(var0 != 0.0, jax.lax.rsqrt(var0), 1.0)
    use_norm = count_f > 1.0
    scale = jnp.where(use_norm, inv_sd, 1.0)
    shift = jnp.where(use_norm, mean.astype(jnp.float32), 0.0) * scale
    scale_r = scale.reshape(1, d_in)
    shift_r = shift.reshape(1, d_in)

    m_mat = jnp.full((d_model, d_model), 1.0 / d_model, dtype=bf)
    k_sel = jnp.zeros((d_in, d_model), jnp.float32).at[0, :].set(1.0).astype(bf)

    weights = [w_emb.astype(bf), b_emb, ln1_w, ln1_b,
               w_ff1.astype(bf), b_ff1, w_ff2.astype(bf), b_ff2,
               ln2_w, ln2_b, m_mat, k_sel]
    weight_specs = [pl.BlockSpec(tuple(w.shape), lambda i: (0, 0))
                    for w in weights]

    out = pl.pallas_call(
        functools.partial(_fused_kernel, cliprange=5.0),
        out_shape=jax.ShapeDtypeStruct((R_pad, d_model), jnp.float32),
        grid=(R_pad // tr,),
        in_specs=[
            pl.BlockSpec((1, d_in), lambda i: (0, 0)),
            pl.BlockSpec((1, d_in), lambda i: (0, 0)),
            pl.BlockSpec((tr, d_in), lambda i: (i, 0)),
        ] + weight_specs,
        out_specs=pl.BlockSpec((tr, d_model), lambda i: (i, 0)),
        compiler_params=pltpu.CompilerParams(
            dimension_semantics=("parallel",),
            vmem_limit_bytes=64 * 1024 * 1024,
        ),
    )(scale_r, shift_r, x2, *weights)

    y = (out if R_pad == R else out[:R]).reshape(B, items, d_model)
    mask = x[:, :, 0] == 0
    return y, None, mask
